# baseline (speedup 1.0000x reference)
import functools

import jax
import jax.numpy as jnp
from jax.experimental import pallas as pl
from jax.experimental.pallas import tpu as pltpu

_BLOCK = 1024


def _bdot(a, b):
    return jax.lax.dot(a.astype(jnp.bfloat16), b,
                       preferred_element_type=jnp.float32)


def _fused_mlp_kernel(x_ref, kt_ref, a0c_ref, a0s_ref, b0_ref, a1_ref, b1_ref,
                      a2_ref, b2_ref, a3_ref, b3_ref, a4_ref, b4_ref, y_ref):
    x = x_ref[...]
    kt = kt_ref[...]
    z = (x[:, 0:1] * kt[0:1, :]
         + x[:, 1:2] * kt[1:2, :]
         + x[:, 2:3] * kt[2:3, :])
    h = jnp.tanh(_bdot(jnp.cos(z), a0c_ref[...])
                 + _bdot(jnp.sin(z), a0s_ref[...]) + b0_ref[...])
    h = jnp.tanh(_bdot(h, a1_ref[...]) + b1_ref[...])
    h = jnp.tanh(_bdot(h, a2_ref[...]) + b2_ref[...])
    h = jnp.tanh(_bdot(h, a3_ref[...]) + b3_ref[...])
    y_ref[...] = _bdot(h, a4_ref[...]) + b4_ref[...]


@jax.jit
def kernel(x, kernel_rff, W0, b0, W1, b1, W2, b2, W3, b3, W4, b4):
    n = x.shape[0]
    d0 = W0.shape[1]
    half = d0 // 2
    kt = kernel_rff.T
    a0 = W0.T
    a0c, a0s = a0[:half], a0[half:]
    a1, a2, a3, a4 = W1.T, W2.T, W3.T, W4.T
    bf = jnp.bfloat16
    a0c, a0s = a0c.astype(bf), a0s.astype(bf)
    a1, a2, a3, a4 = a1.astype(bf), a2.astype(bf), a3.astype(bf), a4.astype(bf)
    grid = (n // _BLOCK,)

    def rows(i):
        return (i, 0)

    def whole(i):
        return (0, 0)

    full = lambda arr: pl.BlockSpec(arr.shape, whole)
    out = pl.pallas_call(
        _fused_mlp_kernel,
        grid=grid,
        in_specs=[
            pl.BlockSpec((_BLOCK, 3), rows),
            full(kt),
            full(a0c), full(a0s), pl.BlockSpec((1, b0.shape[0]), whole),
            full(a1), pl.BlockSpec((1, b1.shape[0]), whole),
            full(a2), pl.BlockSpec((1, b2.shape[0]), whole),
            full(a3), pl.BlockSpec((1, b3.shape[0]), whole),
            full(a4), pl.BlockSpec((1, b4.shape[0]), whole),
        ],
        out_specs=pl.BlockSpec((_BLOCK, 3), rows),
        out_shape=jax.ShapeDtypeStruct((n, 3), x.dtype),
        compiler_params=pltpu.CompilerParams(
            dimension_semantics=("parallel",),
        ),
    )(x, kt, a0c, a0s, b0[None, :], a1, b1[None, :], a2, b2[None, :],
      a3, b3[None, :], a4, b4[None, :])
    return out


# speedup vs baseline: 1.0134x; 1.0134x over previous
import functools

import jax
import jax.numpy as jnp
from jax.experimental import pallas as pl
from jax.experimental.pallas import tpu as pltpu

_BLOCK = 2048


def _bdot(a, b):
    return jax.lax.dot(a.astype(jnp.bfloat16), b,
                       preferred_element_type=jnp.float32)


def _fused_mlp_kernel(x_ref, kt_ref, a0c_ref, a0s_ref, b0_ref, a1_ref, b1_ref,
                      a2_ref, b2_ref, a3_ref, b3_ref, a4_ref, b4_ref, y_ref):
    x = x_ref[...]
    kt = kt_ref[...]
    z = (x[:, 0:1] * kt[0:1, :]
         + x[:, 1:2] * kt[1:2, :]
         + x[:, 2:3] * kt[2:3, :])
    h = jnp.tanh(_bdot(jnp.cos(z), a0c_ref[...])
                 + _bdot(jnp.sin(z), a0s_ref[...]) + b0_ref[...])
    h = jnp.tanh(_bdot(h, a1_ref[...]) + b1_ref[...])
    h = jnp.tanh(_bdot(h, a2_ref[...]) + b2_ref[...])
    h = jnp.tanh(_bdot(h, a3_ref[...]) + b3_ref[...])
    y_ref[...] = _bdot(h, a4_ref[...]) + b4_ref[...]


@jax.jit
def kernel(x, kernel_rff, W0, b0, W1, b1, W2, b2, W3, b3, W4, b4):
    n = x.shape[0]
    d0 = W0.shape[1]
    half = d0 // 2
    kt = kernel_rff.T
    a0 = W0.T
    a0c, a0s = a0[:half], a0[half:]
    a1, a2, a3, a4 = W1.T, W2.T, W3.T, W4.T
    bf = jnp.bfloat16
    a0c, a0s = a0c.astype(bf), a0s.astype(bf)
    a1, a2, a3, a4 = a1.astype(bf), a2.astype(bf), a3.astype(bf), a4.astype(bf)
    grid = (n // _BLOCK,)

    def rows(i):
        return (i, 0)

    def whole(i):
        return (0, 0)

    full = lambda arr: pl.BlockSpec(arr.shape, whole)
    out = pl.pallas_call(
        _fused_mlp_kernel,
        grid=grid,
        in_specs=[
            pl.BlockSpec((_BLOCK, 3), rows),
            full(kt),
            full(a0c), full(a0s), pl.BlockSpec((1, b0.shape[0]), whole),
            full(a1), pl.BlockSpec((1, b1.shape[0]), whole),
            full(a2), pl.BlockSpec((1, b2.shape[0]), whole),
            full(a3), pl.BlockSpec((1, b3.shape[0]), whole),
            full(a4), pl.BlockSpec((1, b4.shape[0]), whole),
        ],
        out_specs=pl.BlockSpec((_BLOCK, 3), rows),
        out_shape=jax.ShapeDtypeStruct((n, 3), x.dtype),
        compiler_params=pltpu.CompilerParams(
            dimension_semantics=("parallel",),
        ),
    )(x, kt, a0c, a0s, b0[None, :], a1, b1[None, :], a2, b2[None, :],
      a3, b3[None, :], a4, b4[None, :])
    return out


# speedup vs baseline: 1.0607x; 1.0467x over previous
import jax
import jax.numpy as jnp
from jax.experimental import pallas as pl
from jax.experimental.pallas import tpu as pltpu

_BLOCK = 2048


def _bdot(a, b):
    return jax.lax.dot(a, b, preferred_element_type=jnp.float32)


def _fused_mlp_kernel(x_ref, kt_ref, a0_ref, b0_ref, a1_ref, b1_ref,
                      a2_ref, b2_ref, a3_ref, b3_ref, a4_ref, b4_ref, y_ref,
                      feats_ref):
    i = pl.program_id(0)
    sel = jax.lax.rem(i, 2)
    osel = 1 - sel

    x = x_ref[...]
    kt = kt_ref[...]
    z = (x[:, 0:1] * kt[0:1, :]
         + x[:, 1:2] * kt[1:2, :]
         + x[:, 2:3] * kt[2:3, :])
    feats_ref[pl.ds(sel, 1), :, 0:128] = jnp.cos(z).astype(jnp.bfloat16)[None]
    feats_ref[pl.ds(sel, 1), :, 128:256] = jnp.sin(z).astype(jnp.bfloat16)[None]

    f = feats_ref[pl.ds(osel, 1), :, :][0]
    h = jnp.tanh(_bdot(f, a0_ref[...]) + b0_ref[...])
    h = jnp.tanh(_bdot(h.astype(jnp.bfloat16), a1_ref[...]) + b1_ref[...])
    h = jnp.tanh(_bdot(h.astype(jnp.bfloat16), a2_ref[...]) + b2_ref[...])
    h = jnp.tanh(_bdot(h.astype(jnp.bfloat16), a3_ref[...]) + b3_ref[...])
    y_ref[...] = _bdot(h.astype(jnp.bfloat16), a4_ref[...]) + b4_ref[...]


@jax.jit
def kernel(x, kernel_rff, W0, b0, W1, b1, W2, b2, W3, b3, W4, b4):
    n = x.shape[0]
    nb = n // _BLOCK
    kt = kernel_rff.T
    bf = jnp.bfloat16
    a0 = W0.T.astype(bf)
    a1, a2, a3, a4 = (W1.T.astype(bf), W2.T.astype(bf), W3.T.astype(bf),
                      W4.T.astype(bf))
    grid = (nb + 1,)

    def rows_in(i):
        return (jnp.minimum(i, nb - 1), 0)

    def rows_out(i):
        return (jnp.maximum(i - 1, 0), 0)

    def whole(i):
        return (0, 0)

    full = lambda arr: pl.BlockSpec(arr.shape, whole)
    out = pl.pallas_call(
        _fused_mlp_kernel,
        grid=grid,
        in_specs=[
            pl.BlockSpec((_BLOCK, 3), rows_in),
            full(kt),
            full(a0), pl.BlockSpec((1, b0.shape[0]), whole),
            full(a1), pl.BlockSpec((1, b1.shape[0]), whole),
            full(a2), pl.BlockSpec((1, b2.shape[0]), whole),
            full(a3), pl.BlockSpec((1, b3.shape[0]), whole),
            full(a4), pl.BlockSpec((1, b4.shape[0]), whole),
        ],
        out_specs=pl.BlockSpec((_BLOCK, 3), rows_out),
        out_shape=jax.ShapeDtypeStruct((n, 3), x.dtype),
        scratch_shapes=[pltpu.VMEM((2, _BLOCK, 256), jnp.bfloat16)],
        compiler_params=pltpu.CompilerParams(
            dimension_semantics=("arbitrary",),
        ),
    )(x, kt, a0, b0[None, :], a1, b1[None, :], a2, b2[None, :],
      a3, b3[None, :], a4, b4[None, :])
    return out
